# baseline (speedup 1.0000x reference)
import jax
import jax.numpy as jnp
from jax.experimental import pallas as pl
from jax.experimental.pallas import tpu as pltpu

NSTRIPE = 8


def _body(x_hbm, o_ref, *scratch):
    bufs = scratch[:NSTRIPE]
    sems = scratch[NSTRIPE]
    B = x_hbm.shape[0]
    rows = B // NSTRIPE
    for s in range(NSTRIPE):
        pltpu.make_async_copy(
            x_hbm.at[pl.ds(s * rows, rows), :],
            bufs[s],
            sems.at[s],
        ).start()
    for s in range(NSTRIPE):
        pltpu.make_async_copy(
            x_hbm.at[pl.ds(s * rows, rows), :],
            bufs[s],
            sems.at[s],
        ).wait()
    o_ref[...] = bufs[0][:8, :]


def kernel(t, x_flat, W1, b1, W2, b2, W3, b3, W4, b4):
    del t
    B, D = x_flat.shape
    rows = B // NSTRIPE
    return pl.pallas_call(
        _body,
        in_specs=[pl.BlockSpec(memory_space=pltpu.MemorySpace.HBM)],
        out_specs=pl.BlockSpec(memory_space=pltpu.MemorySpace.VMEM),
        out_shape=jax.ShapeDtypeStruct((8, D), jnp.float32),
        scratch_shapes=[pltpu.VMEM((rows, D), jnp.float32) for _ in range(NSTRIPE)]
        + [pltpu.SemaphoreType.DMA((NSTRIPE,))],
    )(x_flat).repeat(B // 8, axis=0)


# speedup vs baseline: 1.0315x; 1.0315x over previous
import jax
import jax.numpy as jnp
from jax.experimental import pallas as pl
from jax.experimental.pallas import tpu as pltpu


def _mlp_body(x_ref, w1_ref, b1_ref, w2_ref, b2_ref, w3_ref, b3_ref,
              w4_ref, b4_ref, o_ref):
    h = jnp.dot(x_ref[...], w1_ref[...],
                preferred_element_type=jnp.float32) + b1_ref[...]
    h = jnp.maximum(h, 0.0)
    h = jnp.dot(h.astype(jnp.bfloat16), w2_ref[...],
                preferred_element_type=jnp.float32) + b2_ref[...]
    h = h * jax.nn.sigmoid(h)
    h = jnp.dot(h.astype(jnp.bfloat16), w3_ref[...],
                preferred_element_type=jnp.float32) + b3_ref[...]
    h = h * jax.nn.sigmoid(h)
    h = jnp.dot(h.astype(jnp.bfloat16), w4_ref[...],
                preferred_element_type=jnp.float32) + b4_ref[...]
    o_ref[...] = h.astype(jnp.bfloat16)


def kernel(t, x_flat, W1, b1, W2, b2, W3, b3, W4, b4):
    del t
    B, D = x_flat.shape
    H = W1.shape[1]
    BM = 4096
    grid = (B // BM,)

    def full(shape):
        return pl.BlockSpec(shape, lambda i: (0, 0))

    out = pl.pallas_call(
        _mlp_body,
        grid=grid,
        in_specs=[
            pl.BlockSpec((BM, D), lambda i: (i, 0)),
            full((D, H)), full((1, H)),
            full((H, H)), full((1, H)),
            full((H, H)), full((1, H)),
            full((H, D)), full((1, D)),
        ],
        out_specs=pl.BlockSpec((BM, D), lambda i: (i, 0)),
        out_shape=jax.ShapeDtypeStruct((B, D), jnp.bfloat16),
    )(x_flat.astype(jnp.bfloat16),
      W1.astype(jnp.bfloat16), b1.reshape(1, H),
      W2.astype(jnp.bfloat16), b2.reshape(1, H),
      W3.astype(jnp.bfloat16), b3.reshape(1, H),
      W4.astype(jnp.bfloat16), b4.reshape(1, D))
    return out.astype(jnp.float32)
